# baseline (speedup 1.0000x reference)
import functools

import jax
import jax.numpy as jnp
import numpy as np
from jax import lax
from jax.experimental import pallas as pl
from jax.experimental.pallas import tpu as pltpu
from jax.experimental.pallas import tpu_sc as plsc

NC = 2
NS = 16
L = 16


def _matmul_body(x_ref, w_ref, o_ref):
    o_ref[0] = jnp.dot(x_ref[...], w_ref[0], preferred_element_type=jnp.float32)


def _project(x, w_stacked, bm):
    n, d = x.shape
    return pl.pallas_call(
        _matmul_body,
        grid=(2, n // bm),
        in_specs=[
            pl.BlockSpec((bm, d), lambda r, i: (i, 0)),
            pl.BlockSpec((1, d, d), lambda r, i: (r, 0, 0)),
        ],
        out_specs=pl.BlockSpec((1, bm, d), lambda r, i: (r, i, 0)),
        out_shape=jax.ShapeDtypeStruct((2, n, d), jnp.float32),
    )(x, w_stacked)


def _combine_body(p_ref, o_ref):
    o_ref[...] = jnp.maximum(p_ref[0] + p_ref[1], 0.0)


def _combine(partials, n, bm):
    _, _, d = partials.shape
    return pl.pallas_call(
        _combine_body,
        grid=(n // bm,),
        in_specs=[pl.BlockSpec((2, bm, d), lambda i: (0, i, 0))],
        out_specs=pl.BlockSpec((bm, d), lambda i: (i, 0)),
        out_shape=jax.ShapeDtypeStruct((n, d), jnp.float32),
    )(partials)


def _spmm_body(n_pad, d, ngroup, gsz, ch,
               table, srcs, dsts, vals, out,
               src_v, dst_v, vals_v, rows_v, acc, gsem, ssem):
    c = lax.axis_index("c")
    s = lax.axis_index("s")

    def _zero_row(i, carry):
        for q in range(d // L):
            rows_v[0, i, pl.ds(q * L, L)] = jnp.zeros((L,), jnp.float32)
        return carry

    lax.fori_loop(0, ch, _zero_row, 0)
    rows_per_tile = n_pad // NS
    base_row = s * rows_per_tile
    for k in range(rows_per_tile // ch):
        pltpu.sync_copy(rows_v.at[0], acc.at[pl.ds(base_row + k * ch, ch)])
    plsc.subcore_barrier()

    def _scale(b, gi):
        def _16edges(g, carry2):
            v16 = vals_v[gi, pl.ds(g * L, L)]
            for j in range(L):
                e = g * L + j
                vb = jnp.full((L,), v16[j], jnp.float32)
                for q in range(d // L):
                    rows_v[b, e, pl.ds(q * L, L)] = (
                        rows_v[b, e, pl.ds(q * L, L)] * vb)
            return carry2

        lax.fori_loop(0, ch // L, _16edges, 0)

    qs = [q // 8 * 8 for q in (0, ch // 4, ch // 2, 3 * ch // 4)] + [ch]

    def _gather4(b, gi):
        descs = []
        for i4 in range(4):
            lo, hhi = qs[i4], qs[i4 + 1]
            descs.append(pltpu.async_copy(
                table.at[src_v.at[gi, pl.ds(lo, hhi - lo)]],
                rows_v.at[b, pl.ds(lo, hhi - lo)],
                ssem if i4 % 2 == 0 else gsem))
        return descs

    def _group_of_chunks(og, carry):
        pltpu.sync_copy(srcs.at[c, s, og], src_v)
        pltpu.sync_copy(dsts.at[c, s, og], dst_v)
        pltpu.sync_copy(vals.at[c, s, og], vals_v)
        pltpu.async_copy(table.at[src_v.at[0]], rows_v.at[0], gsem).wait()

        def _pair(i2, carry1):
            k = 2 * i2
            gbs = _gather4(1, k + 1)
            _scale(0, k)
            pltpu.sync_copy(rows_v.at[0], acc.at[dst_v.at[k]], add=True)
            for g in gbs:
                g.wait()

            @pl.when(k + 2 < gsz)
            def _():
                gas = _gather4(0, k + 2)
                _scale(1, k + 1)
                pltpu.sync_copy(rows_v.at[1], acc.at[dst_v.at[k + 1]], add=True)
                for g in gas:
                    g.wait()

            @pl.when(k + 2 >= gsz)
            def _():
                _scale(1, k + 1)
                pltpu.sync_copy(rows_v.at[1], acc.at[dst_v.at[k + 1]], add=True)

            return carry1

        lax.fori_loop(0, gsz // 2, _pair, 0)
        return carry

    lax.fori_loop(0, ngroup, _group_of_chunks, 0)
    plsc.subcore_barrier()

    pltpu.sync_copy(
        acc.at[pl.ds(base_row, rows_per_tile)],
        out.at[c, pl.ds(base_row, rows_per_tile)],
    )


def _spmm(table, srcs, dsts, vals, n_pad, d, ngroup, gsz, ch):
    mesh = plsc.VectorSubcoreMesh(
        core_axis_name="c", subcore_axis_name="s", num_cores=NC, num_subcores=NS
    )
    body = functools.partial(_spmm_body, n_pad, d, ngroup, gsz, ch)
    return pl.kernel(
        body,
        out_type=jax.ShapeDtypeStruct((NC, n_pad, d), jnp.float32),
        mesh=mesh,
        scratch_types=[
            pltpu.VMEM((gsz, ch), jnp.int32),
            pltpu.VMEM((gsz, ch), jnp.int32),
            pltpu.VMEM((gsz, ch), jnp.float32),
            pltpu.VMEM((2, ch, d), jnp.float32),
            pltpu.VMEM_SHARED((n_pad, d), jnp.float32),
            pltpu.SemaphoreType.DMA,
            pltpu.SemaphoreType.DMA,
        ],
    )(table, srcs, dsts, vals)


def kernel(x, edge_index_1, edge_vals_1, edge_index_2, edge_vals_2,
           weights_1, weights_2):
    n, d = x.shape
    e = edge_vals_1.shape[0]

    ept = e // NS
    ch = max(cc for cc in range(L, 129, L) if ept % cc == 0)
    nchunk = ept // ch
    gsz = max(g for g in range(2, 65, 2) if nchunk % g == 0)
    ngroup = nchunk // gsz
    rows_per_tile = -(-(n // NS) // (8 * ch)) * (8 * ch)
    n_pad = NS * rows_per_tile

    table = _project(x, jnp.stack([weights_1, weights_2]), bm=1000)
    table = table.reshape(2 * n, d)

    srcs = jnp.stack([edge_index_1[1], edge_index_2[1] + n]).reshape(
        NC, NS, ngroup, gsz, ch)
    dsts = jnp.stack([edge_index_1[0], edge_index_2[0]]).reshape(
        NC, NS, ngroup, gsz, ch)
    vals = jnp.stack([edge_vals_1, edge_vals_2]).reshape(NC, NS, ngroup, gsz, ch)

    partials = _spmm(table, srcs, dsts, vals, n_pad, d, ngroup, gsz, ch)
    return _combine(partials, n, bm=1000)


# speedup vs baseline: 1.0064x; 1.0064x over previous
import functools

import jax
import jax.numpy as jnp
import numpy as np
from jax import lax
from jax.experimental import pallas as pl
from jax.experimental.pallas import tpu as pltpu
from jax.experimental.pallas import tpu_sc as plsc

NC = 2
NS = 16
L = 16


def _matmul_body(x_ref, w_ref, o_ref):
    o_ref[0] = jnp.dot(x_ref[...], w_ref[0], preferred_element_type=jnp.float32)


def _project(x, w_stacked, bm):
    n, d = x.shape
    return pl.pallas_call(
        _matmul_body,
        grid=(2, n // bm),
        in_specs=[
            pl.BlockSpec((bm, d), lambda r, i: (i, 0)),
            pl.BlockSpec((1, d, d), lambda r, i: (r, 0, 0)),
        ],
        out_specs=pl.BlockSpec((1, bm, d), lambda r, i: (r, i, 0)),
        out_shape=jax.ShapeDtypeStruct((2, n, d), jnp.float32),
    )(x, w_stacked)


def _combine_body(p_ref, o_ref):
    o_ref[...] = jnp.maximum(p_ref[0] + p_ref[1], 0.0)


def _combine(partials, n, bm):
    _, _, d = partials.shape
    return pl.pallas_call(
        _combine_body,
        grid=(n // bm,),
        in_specs=[pl.BlockSpec((2, bm, d), lambda i: (0, i, 0))],
        out_specs=pl.BlockSpec((bm, d), lambda i: (i, 0)),
        out_shape=jax.ShapeDtypeStruct((n, d), jnp.float32),
    )(partials)


def _spmm_body(n_pad, d, ngroup, gsz, ch,
               table, srcs, dsts, vals, out,
               src_v, dst_v, vals_v, rows_v, acc, gsem, ssem):
    c = lax.axis_index("c")
    s = lax.axis_index("s")

    def _zero_row(i, carry):
        for q in range(d // L):
            rows_v[0, i, pl.ds(q * L, L)] = jnp.zeros((L,), jnp.float32)
        return carry

    lax.fori_loop(0, ch, _zero_row, 0)
    rows_per_tile = n_pad // NS
    base_row = s * rows_per_tile
    for k in range(rows_per_tile // ch):
        pltpu.sync_copy(rows_v.at[0], acc.at[pl.ds(base_row + k * ch, ch)])
    plsc.subcore_barrier()

    def _scale(b, gi):
        def _16edges(g, carry2):
            v16 = vals_v[gi, pl.ds(g * L, L)]
            for j in range(L):
                e = g * L + j
                vb = jnp.full((L,), v16[j], jnp.float32)
                for q in range(d // L):
                    rows_v[b, e, pl.ds(q * L, L)] = (
                        rows_v[b, e, pl.ds(q * L, L)] * vb)
            return carry2

        lax.fori_loop(0, ch // L, _16edges, 0)

    h2 = ch // 2 // 8 * 8

    def _gather2(b, gi):
        g1 = pltpu.async_copy(
            table.at[src_v.at[gi, pl.ds(0, h2)]],
            rows_v.at[b, pl.ds(0, h2)], ssem)
        g2 = pltpu.async_copy(
            table.at[src_v.at[gi, pl.ds(h2, ch - h2)]],
            rows_v.at[b, pl.ds(h2, ch - h2)], gsem)
        return (g1, g2)

    def _group_of_chunks(og, carry):
        pltpu.sync_copy(srcs.at[c, s, og], src_v)
        pltpu.sync_copy(dsts.at[c, s, og], dst_v)
        pltpu.sync_copy(vals.at[c, s, og], vals_v)
        pltpu.async_copy(table.at[src_v.at[0]], rows_v.at[0], gsem).wait()

        def _pair(i2, carry1):
            k = 2 * i2
            gbs = _gather2(1, k + 1)
            _scale(0, k)
            pltpu.sync_copy(rows_v.at[0], acc.at[dst_v.at[k]], add=True)
            for g in gbs:
                g.wait()

            @pl.when(k + 2 < gsz)
            def _():
                gas = _gather2(0, k + 2)
                _scale(1, k + 1)
                pltpu.sync_copy(rows_v.at[1], acc.at[dst_v.at[k + 1]], add=True)
                for g in gas:
                    g.wait()

            @pl.when(k + 2 >= gsz)
            def _():
                _scale(1, k + 1)
                pltpu.sync_copy(rows_v.at[1], acc.at[dst_v.at[k + 1]], add=True)

            return carry1

        lax.fori_loop(0, gsz // 2, _pair, 0)
        return carry

    lax.fori_loop(0, ngroup, _group_of_chunks, 0)
    plsc.subcore_barrier()

    pltpu.sync_copy(
        acc.at[pl.ds(base_row, rows_per_tile)],
        out.at[c, pl.ds(base_row, rows_per_tile)],
    )


def _spmm(table, srcs, dsts, vals, n_pad, d, ngroup, gsz, ch):
    mesh = plsc.VectorSubcoreMesh(
        core_axis_name="c", subcore_axis_name="s", num_cores=NC, num_subcores=NS
    )
    body = functools.partial(_spmm_body, n_pad, d, ngroup, gsz, ch)
    return pl.kernel(
        body,
        out_type=jax.ShapeDtypeStruct((NC, n_pad, d), jnp.float32),
        mesh=mesh,
        scratch_types=[
            pltpu.VMEM((gsz, ch), jnp.int32),
            pltpu.VMEM((gsz, ch), jnp.int32),
            pltpu.VMEM((gsz, ch), jnp.float32),
            pltpu.VMEM((2, ch, d), jnp.float32),
            pltpu.VMEM_SHARED((n_pad, d), jnp.float32),
            pltpu.SemaphoreType.DMA,
            pltpu.SemaphoreType.DMA,
        ],
    )(table, srcs, dsts, vals)


def kernel(x, edge_index_1, edge_vals_1, edge_index_2, edge_vals_2,
           weights_1, weights_2):
    n, d = x.shape
    e = edge_vals_1.shape[0]

    ept = e // NS
    ch = max(cc for cc in range(L, 129, L) if ept % cc == 0)
    nchunk = ept // ch
    gsz = max(g for g in range(2, 65, 2) if nchunk % g == 0)
    ngroup = nchunk // gsz
    rows_per_tile = -(-(n // NS) // (8 * ch)) * (8 * ch)
    n_pad = NS * rows_per_tile

    table = _project(x, jnp.stack([weights_1, weights_2]), bm=1000)
    table = table.reshape(2 * n, d)

    srcs = jnp.stack([edge_index_1[1], edge_index_2[1] + n]).reshape(
        NC, NS, ngroup, gsz, ch)
    dsts = jnp.stack([edge_index_1[0], edge_index_2[0]]).reshape(
        NC, NS, ngroup, gsz, ch)
    vals = jnp.stack([edge_vals_1, edge_vals_2]).reshape(NC, NS, ngroup, gsz, ch)

    partials = _spmm(table, srcs, dsts, vals, n_pad, d, ngroup, gsz, ch)
    return _combine(partials, n, bm=1000)


# speedup vs baseline: 1.0184x; 1.0120x over previous
import functools

import jax
import jax.numpy as jnp
import numpy as np
from jax import lax
from jax.experimental import pallas as pl
from jax.experimental.pallas import tpu as pltpu
from jax.experimental.pallas import tpu_sc as plsc

NC = 2
NS = 16
L = 16


def _matmul_body(x_ref, w_ref, o_ref):
    o_ref[0] = jnp.dot(x_ref[...], w_ref[0], preferred_element_type=jnp.float32)


def _project(x, w_stacked, bm):
    n, d = x.shape
    return pl.pallas_call(
        _matmul_body,
        grid=(2, n // bm),
        in_specs=[
            pl.BlockSpec((bm, d), lambda r, i: (i, 0)),
            pl.BlockSpec((1, d, d), lambda r, i: (r, 0, 0)),
        ],
        out_specs=pl.BlockSpec((1, bm, d), lambda r, i: (r, i, 0)),
        out_shape=jax.ShapeDtypeStruct((2, n, d), jnp.float32),
    )(x, w_stacked)


def _combine_body(p_ref, o_ref):
    o_ref[...] = jnp.maximum(p_ref[0] + p_ref[1], 0.0)


def _combine(partials, n, bm):
    _, _, d = partials.shape
    return pl.pallas_call(
        _combine_body,
        grid=(n // bm,),
        in_specs=[pl.BlockSpec((2, bm, d), lambda i: (0, i, 0))],
        out_specs=pl.BlockSpec((bm, d), lambda i: (i, 0)),
        out_shape=jax.ShapeDtypeStruct((n, d), jnp.float32),
    )(partials)


def _spmm_body(n_pad, d, ngroup, gsz, ch,
               table, srcs1, dsts1, vals1, srcs2, dsts2, vals2, out,
               src_v, dst_v, vals_v, rows_v, acc, gsem, ssem):
    c = lax.axis_index("c")
    s = lax.axis_index("s")

    def _zero_row(i, carry):
        for q in range(d // L):
            rows_v[0, i, pl.ds(q * L, L)] = jnp.zeros((L,), jnp.float32)
        return carry

    lax.fori_loop(0, ch, _zero_row, 0)
    rows_per_tile = n_pad // NS
    base_row = s * rows_per_tile
    for k in range(rows_per_tile // ch):
        pltpu.sync_copy(rows_v.at[0], acc.at[pl.ds(base_row + k * ch, ch)])
    plsc.subcore_barrier()

    def _scale(b, gi):
        def _16edges(g, carry2):
            v16 = vals_v[gi, pl.ds(g * L, L)]
            for j in range(L):
                e = g * L + j
                vb = jnp.full((L,), v16[j], jnp.float32)
                for q in range(d // L):
                    rows_v[b, e, pl.ds(q * L, L)] = (
                        rows_v[b, e, pl.ds(q * L, L)] * vb)
            return carry2

        lax.fori_loop(0, ch // L, _16edges, 0)

    h2 = ch // 2 // 8 * 8

    def _gather2(b, gi):
        g1 = pltpu.async_copy(
            table.at[c].at[src_v.at[gi, pl.ds(0, h2)]],
            rows_v.at[b, pl.ds(0, h2)], ssem)
        g2 = pltpu.async_copy(
            table.at[c].at[src_v.at[gi, pl.ds(h2, ch - h2)]],
            rows_v.at[b, pl.ds(h2, ch - h2)], gsem)
        return (g1, g2)

    def _group_of_chunks(og, carry):
        @pl.when(c == 0)
        def _():
            pltpu.sync_copy(srcs1.at[s, og], src_v)
            pltpu.sync_copy(dsts1.at[s, og], dst_v)
            pltpu.sync_copy(vals1.at[s, og], vals_v)

        @pl.when(c == 1)
        def _():
            pltpu.sync_copy(srcs2.at[s, og], src_v)
            pltpu.sync_copy(dsts2.at[s, og], dst_v)
            pltpu.sync_copy(vals2.at[s, og], vals_v)

        pltpu.async_copy(table.at[c].at[src_v.at[0]], rows_v.at[0],
                         gsem).wait()

        def _pair(i2, carry1):
            k = 2 * i2
            gbs = _gather2(1, k + 1)
            _scale(0, k)
            pltpu.sync_copy(rows_v.at[0], acc.at[dst_v.at[k]], add=True)
            for g in gbs:
                g.wait()

            @pl.when(k + 2 < gsz)
            def _():
                gas = _gather2(0, k + 2)
                _scale(1, k + 1)
                pltpu.sync_copy(rows_v.at[1], acc.at[dst_v.at[k + 1]], add=True)
                for g in gas:
                    g.wait()

            @pl.when(k + 2 >= gsz)
            def _():
                _scale(1, k + 1)
                pltpu.sync_copy(rows_v.at[1], acc.at[dst_v.at[k + 1]], add=True)

            return carry1

        lax.fori_loop(0, gsz // 2, _pair, 0)
        return carry

    lax.fori_loop(0, ngroup, _group_of_chunks, 0)
    plsc.subcore_barrier()

    pltpu.sync_copy(
        acc.at[pl.ds(base_row, rows_per_tile)],
        out.at[c, pl.ds(base_row, rows_per_tile)],
    )


def _spmm(table, edges, n_pad, d, ngroup, gsz, ch):
    mesh = plsc.VectorSubcoreMesh(
        core_axis_name="c", subcore_axis_name="s", num_cores=NC, num_subcores=NS
    )
    body = functools.partial(_spmm_body, n_pad, d, ngroup, gsz, ch)
    return pl.kernel(
        body,
        out_type=jax.ShapeDtypeStruct((NC, n_pad, d), jnp.float32),
        mesh=mesh,
        scratch_types=[
            pltpu.VMEM((gsz, ch), jnp.int32),
            pltpu.VMEM((gsz, ch), jnp.int32),
            pltpu.VMEM((gsz, ch), jnp.float32),
            pltpu.VMEM((2, ch, d), jnp.float32),
            pltpu.VMEM_SHARED((n_pad, d), jnp.float32),
            pltpu.SemaphoreType.DMA,
            pltpu.SemaphoreType.DMA,
        ],
    )(table, *edges)


def kernel(x, edge_index_1, edge_vals_1, edge_index_2, edge_vals_2,
           weights_1, weights_2):
    n, d = x.shape
    e = edge_vals_1.shape[0]

    ept = e // NS
    ch = max(cc for cc in range(L, 129, L) if ept % cc == 0)
    nchunk = ept // ch
    gsz = max(g for g in range(2, 65, 2) if nchunk % g == 0)
    ngroup = nchunk // gsz
    rows_per_tile = -(-(n // NS) // (8 * ch)) * (8 * ch)
    n_pad = NS * rows_per_tile

    table = _project(x, jnp.stack([weights_1, weights_2]), bm=1000)

    esh = (NS, ngroup, gsz, ch)
    edges = (
        edge_index_1[1].reshape(esh), edge_index_1[0].reshape(esh),
        edge_vals_1.reshape(esh),
        edge_index_2[1].reshape(esh), edge_index_2[0].reshape(esh),
        edge_vals_2.reshape(esh),
    )
    partials = _spmm(table, edges, n_pad, d, ngroup, gsz, ch)
    return _combine(partials, n, bm=1000)


# speedup vs baseline: 1.0203x; 1.0019x over previous
import functools

import jax
import jax.numpy as jnp
from jax import lax
from jax.experimental import pallas as pl
from jax.experimental.pallas import tpu as pltpu
from jax.experimental.pallas import tpu_sc as plsc

NC = 2
NS = 16
L = 16


def _matmul_body(x_ref, w_ref, o_ref):
    o_ref[0] = jnp.dot(x_ref[...], w_ref[0], preferred_element_type=jnp.float32)


def _project(x, w_stacked, bm):
    n, d = x.shape
    return pl.pallas_call(
        _matmul_body,
        grid=(2, n // bm),
        in_specs=[
            pl.BlockSpec((bm, d), lambda r, i: (i, 0)),
            pl.BlockSpec((1, d, d), lambda r, i: (r, 0, 0)),
        ],
        out_specs=pl.BlockSpec((1, bm, d), lambda r, i: (r, i, 0)),
        out_shape=jax.ShapeDtypeStruct((2, n, d), jnp.float32),
    )(x, w_stacked)


def _combine_body(p_ref, o_ref):
    o_ref[...] = jnp.maximum(p_ref[0] + p_ref[1], 0.0)


def _combine(partials, n, bm):
    _, _, d = partials.shape
    return pl.pallas_call(
        _combine_body,
        grid=(n // bm,),
        in_specs=[pl.BlockSpec((2, bm, d), lambda i: (0, i, 0))],
        out_specs=pl.BlockSpec((bm, d), lambda i: (i, 0)),
        out_shape=jax.ShapeDtypeStruct((n, d), jnp.float32),
    )(partials)


def _spmm_body(n_pad, d, ngroup, gsz, ch,
               table, srcs1, dsts1, vals1, srcs2, dsts2, vals2, out,
               src_v, dst_v, vals_v, rows_v, acc, gsem, ssem):
    c = lax.axis_index("c")
    s = lax.axis_index("s")

    def _zero_row(i, carry):
        for q in range(d // L):
            rows_v[0, i, pl.ds(q * L, L)] = jnp.zeros((L,), jnp.float32)
        return carry

    lax.fori_loop(0, ch, _zero_row, 0)
    rows_per_tile = n_pad // NS
    base_row = s * rows_per_tile
    for k in range(rows_per_tile // ch):
        pltpu.sync_copy(rows_v.at[0], acc.at[pl.ds(base_row + k * ch, ch)])
    plsc.subcore_barrier()

    def _scale(b, gi):
        def _16edges(g, carry2):
            v16 = vals_v[gi, pl.ds(g * L, L)]
            for j in range(L):
                e = g * L + j
                vb = jnp.full((L,), v16[j], jnp.float32)
                for q in range(d // L):
                    rows_v[b, e, pl.ds(q * L, L)] = (
                        rows_v[b, e, pl.ds(q * L, L)] * vb)
            return carry2

        lax.fori_loop(0, ch // L, _16edges, 0)

    h2 = ch // 2 // 8 * 8

    def _gather2(b, gi):
        g1 = pltpu.async_copy(
            table.at[c].at[src_v.at[gi, pl.ds(0, h2)]],
            rows_v.at[b, pl.ds(0, h2)], ssem)
        g2 = pltpu.async_copy(
            table.at[c].at[src_v.at[gi, pl.ds(h2, ch - h2)]],
            rows_v.at[b, pl.ds(h2, ch - h2)], gsem)
        return (g1, g2)

    def _group_of_chunks(og, carry):
        @pl.when(c == 0)
        def _():
            pltpu.sync_copy(srcs1.at[s, og], src_v)
            pltpu.sync_copy(dsts1.at[s, og], dst_v)
            pltpu.sync_copy(vals1.at[s, og], vals_v)

        @pl.when(c == 1)
        def _():
            pltpu.sync_copy(srcs2.at[s, og], src_v)
            pltpu.sync_copy(dsts2.at[s, og], dst_v)
            pltpu.sync_copy(vals2.at[s, og], vals_v)

        pltpu.async_copy(table.at[c].at[src_v.at[0]], rows_v.at[0],
                         gsem).wait()

        def _pair(i2, carry1):
            k = 2 * i2
            gbs = _gather2(1, k + 1)
            _scale(0, k)
            pltpu.sync_copy(rows_v.at[0], acc.at[dst_v.at[k]], add=True)
            for g in gbs:
                g.wait()

            @pl.when(k + 2 < gsz)
            def _():
                gas = _gather2(0, k + 2)
                _scale(1, k + 1)
                pltpu.sync_copy(rows_v.at[1], acc.at[dst_v.at[k + 1]], add=True)
                for g in gas:
                    g.wait()

            @pl.when(k + 2 >= gsz)
            def _():
                _scale(1, k + 1)
                pltpu.sync_copy(rows_v.at[1], acc.at[dst_v.at[k + 1]], add=True)

            return carry1

        lax.fori_loop(0, gsz // 2, _pair, 0)
        return carry

    lax.fori_loop(0, ngroup, _group_of_chunks, 0)
    plsc.subcore_barrier()

    pltpu.sync_copy(
        acc.at[pl.ds(base_row, rows_per_tile)],
        out.at[c, pl.ds(base_row, rows_per_tile)],
    )


def _spmm(table, edges, n_pad, d, ngroup, gsz, ch):
    mesh = plsc.VectorSubcoreMesh(
        core_axis_name="c", subcore_axis_name="s", num_cores=NC, num_subcores=NS
    )
    body = functools.partial(_spmm_body, n_pad, d, ngroup, gsz, ch)
    return pl.kernel(
        body,
        out_type=jax.ShapeDtypeStruct((NC, n_pad, d), jnp.float32),
        mesh=mesh,
        scratch_types=[
            pltpu.VMEM((gsz, ch), jnp.int32),
            pltpu.VMEM((gsz, ch), jnp.int32),
            pltpu.VMEM((gsz, ch), jnp.float32),
            pltpu.VMEM((2, ch, d), jnp.float32),
            pltpu.VMEM_SHARED((n_pad, d), jnp.float32),
            pltpu.SemaphoreType.DMA,
            pltpu.SemaphoreType.DMA,
        ],
    )(table, *edges)


def kernel(x, edge_index_1, edge_vals_1, edge_index_2, edge_vals_2,
           weights_1, weights_2):
    n, d = x.shape
    e = edge_vals_1.shape[0]

    ept = e // NS
    ch = max(cc for cc in range(L, 129, L) if ept % cc == 0)
    nchunk = ept // ch
    gsz = max(g for g in range(2, 65, 2) if nchunk % g == 0)
    ngroup = nchunk // gsz
    rows_per_tile = -(-(n // NS) // (8 * ch)) * (8 * ch)
    n_pad = NS * rows_per_tile

    table = _project(x, jnp.stack([weights_1, weights_2]), bm=1000)

    esh = (NS, ngroup, gsz, ch)
    edges = (
        edge_index_1[1].reshape(esh), edge_index_1[0].reshape(esh),
        edge_vals_1.reshape(esh),
        edge_index_2[1].reshape(esh), edge_index_2[0].reshape(esh),
        edge_vals_2.reshape(esh),
    )
    partials = _spmm(table, edges, n_pad, d, ngroup, gsz, ch)
    return _combine(partials, n, bm=1000)
